# baseline (speedup 1.0000x reference)
import jax
import jax.numpy as jnp
from jax import lax
from jax.experimental import pallas as pl
from jax.experimental.pallas import tpu as pltpu
from jax.experimental.pallas import tpu_sc as plsc

B = 16384
D = 64
VOCAB = 1000000
NBLK = (VOCAB + 127) // 128
NC = 2
NS = 16
NW = NC * NS
WIN = 384
OB = 1024
NBUF = 8

_mesh = plsc.VectorSubcoreMesh(core_axis_name="c", subcore_axis_name="s",
                               num_cores=NC, num_subcores=NS)


def _mul(x, n):
    return pl.multiple_of(x, n)


def _sc_body(xs_hbm, sids_hbm, bounds_hbm, Vt_hbm, out_hbm,
             swin, xwin, bv, buf0, buf1, buf2, buf3, buf4, buf5, buf6, buf7, S, obuf,
             sem0, sem1, sem2, sem3, sem4, sem5, sem6, sem7, osem):
    wid = lax.axis_index("s") * NC + lax.axis_index("c")
    qlo = (NBLK * wid) // NW
    qhi = (NBLK * (wid + 1)) // NW
    nq = qhi - qlo

    lanes = lax.iota(jnp.int32, 16)
    flanes = lanes.astype(jnp.float32) * 0.0
    zeros16 = jnp.zeros((16,), jnp.int32)
    fzeros = jnp.zeros((16,), jnp.float32)
    cvecs = [lanes + j * 16 for j in range(D // 16)]

    pltpu.sync_copy(bounds_hbm, bv)

    def sx(ref, pos):
        c = ref[pl.ds(_mul(lax.shift_left(lax.shift_right_logical(pos, 4), 4), 16), 16)]
        return jnp.sum(jnp.where(lanes == lax.bitwise_and(pos, 15), c, zeros16))

    lo = sx(bv, wid)
    hi = sx(bv, wid + 1)

    for k in range(OB // 16):
        obuf[pl.ds(k * 16, 16)] = fzeros
    zcps = [
        pltpu.async_copy(obuf, out_hbm.at[wid, pl.ds(k * OB, OB)], osem)
        for k in range(B // OB)
    ]
    for z in zcps:
        z.wait()

    win0 = _mul(jnp.minimum(lax.shift_left(lax.shift_right_logical(lo, 3), 3), B - WIN), 8)
    pltpu.sync_copy(sids_hbm.at[pl.ds(win0, WIN)], swin)
    pltpu.sync_copy(xs_hbm.at[pl.ds(win0, WIN)], xwin)

    bufs = (buf0, buf1, buf2, buf3, buf4, buf5, buf6, buf7)
    sems = (sem0, sem1, sem2, sem3, sem4, sem5, sem6, sem7)

    def fetch(qidx, b):
        qf = jnp.minimum(qidx, NBLK - 1)
        off = _mul(qf * 128, 128)
        pltpu.async_copy(Vt_hbm.at[:, pl.ds(off, 128)], bufs[b], sems[b])

    for b in range(NBUF):
        fetch(qlo + b, b)

    def wait(b):
        pltpu.make_async_copy(Vt_hbm.at[:, pl.ds(0, 128)], bufs[b], sems[b]).wait()

    def group_emit(p_last):
        g0 = lax.shift_left(lax.shift_right_logical(p_last, 4), 4)
        acc = plsc.load_gather(S, [lanes, zeros16])
        for cc in range(1, 16):
            acc = acc + plsc.load_gather(S, [lanes, zeros16 + cc])
        sig = 1.0 / (1.0 + jnp.exp(-acc))
        gpos = g0 + lanes
        vmask = jnp.logical_and(gpos >= lo, gpos < hi)
        sig = jnp.where(vmask, sig, flanes)
        obuf[pl.ds(_mul(lax.bitwise_and(g0, OB - 1), 16), 16)] = sig

    def flush(p_last):
        fb = _mul(lax.shift_left(lax.shift_right_logical(p_last, 10), 10), 1024)
        pltpu.async_copy(obuf, out_hbm.at[wid, pl.ds(fb, OB)], osem).wait()
        for k in range(OB // 16):
            obuf[pl.ds(k * 16, 16)] = fzeros

    def run_block(q, b, carry):
        buf = bufs[b]

        def cond(c):
            p, win, s = c
            return jnp.logical_and(p < hi, lax.shift_right_logical(s, 7) == q)

        def body(c):
            p, win, s = c
            off = p - win
            m = lax.bitwise_and(s, 127)
            msplat = zeros16 + m
            pv = plsc.load_gather(buf, [cvecs[0], msplat]) * xwin[off, pl.ds(0, 16)]
            for j in range(1, D // 16):
                pv = pv + plsc.load_gather(buf, [cvecs[j], msplat]) * xwin[off, pl.ds(j * 16, 16)]
            S[lax.bitwise_and(p, 15), pl.ds(0, 16)] = pv

            @pl.when(lax.bitwise_and(p, 15) == 15)
            def _():
                group_emit(p)

                @pl.when(lax.bitwise_and(p, OB - 1) == OB - 1)
                def _():
                    flush(p)

            p2 = p + 1
            need = (p2 - win) >= WIN
            nw = _mul(jnp.minimum(lax.shift_left(lax.shift_right_logical(p2, 3), 3), B - WIN), 8)

            @pl.when(need)
            def _():
                pltpu.sync_copy(sids_hbm.at[pl.ds(nw, WIN)], swin)
                pltpu.sync_copy(xs_hbm.at[pl.ds(nw, WIN)], xwin)

            win2 = jnp.where(need, nw, win)
            s2 = sx(swin, jnp.minimum(p2 - win2, WIN - 1))
            return (p2, win2, s2)

        return lax.while_loop(cond, body, carry)

    def outer(o, carry):
        for b in range(NBUF):
            k = o * NBUF + b
            wait(b)
            carry = run_block(qlo + k, b, carry)
            fetch(qlo + k + NBUF, b)
        return carry

    s0 = sx(swin, jnp.minimum(lo - win0, WIN - 1))
    nqo = lax.div(nq + NBUF - 1, jnp.int32(NBUF))
    p, win, _s = lax.fori_loop(0, nqo, outer, (lo, win0, s0))
    for b in range(NBUF):
        wait(b)

    @pl.when(jnp.logical_and(hi > lo, lax.bitwise_and(hi, 15) != 0))
    def _():
        group_emit(hi - 1)

    @pl.when(jnp.logical_and(hi > lo, lax.bitwise_and(hi, OB - 1) != 0))
    def _():
        flush(hi - 1)


_sc_kernel = pl.kernel(
    _sc_body,
    out_type=jax.ShapeDtypeStruct((NW, B), jnp.float32),
    mesh=_mesh,
    compiler_params=pltpu.CompilerParams(needs_layout_passes=False),
    scratch_types=[
        pltpu.VMEM((WIN,), jnp.int32),
        pltpu.VMEM((WIN, D), jnp.float32),
        pltpu.VMEM((64,), jnp.int32),
        pltpu.VMEM((D, 128), jnp.float32),
        pltpu.VMEM((D, 128), jnp.float32),
        pltpu.VMEM((D, 128), jnp.float32),
        pltpu.VMEM((D, 128), jnp.float32),
        pltpu.VMEM((D, 128), jnp.float32),
        pltpu.VMEM((D, 128), jnp.float32),
        pltpu.VMEM((D, 128), jnp.float32),
        pltpu.VMEM((D, 128), jnp.float32),
        pltpu.VMEM((16, 17), jnp.float32),
        pltpu.VMEM((OB,), jnp.float32),
        pltpu.SemaphoreType.DMA,
        pltpu.SemaphoreType.DMA,
        pltpu.SemaphoreType.DMA,
        pltpu.SemaphoreType.DMA,
        pltpu.SemaphoreType.DMA,
        pltpu.SemaphoreType.DMA,
        pltpu.SemaphoreType.DMA,
        pltpu.SemaphoreType.DMA,
        pltpu.SemaphoreType.DMA,
    ],
)


@jax.jit
def kernel(x, emoji_ids, V):
    ids = emoji_ids.astype(jnp.int32)
    iot = jnp.arange(B, dtype=jnp.int32)
    sids, perm = lax.sort_key_val(ids, iot)
    _, inv_perm = lax.sort_key_val(perm, iot)
    x_s = jnp.take(x, perm, axis=0)
    qsplit = jnp.array([((NBLK * w) // NW) * 128 for w in range(NW + 1)],
                       dtype=jnp.int32)
    bounds = jnp.sum(qsplit[:, None] > sids[None, :], axis=1,
                      dtype=jnp.int32)
    boundsp = jnp.zeros((64,), jnp.int32).at[: NW + 1].set(bounds)
    out_p = _sc_kernel(x_s, sids, boundsp, V.T)
    out_s = jnp.sum(out_p, axis=0)
    return jnp.take(out_s, inv_perm)


# speedup vs baseline: 1.1266x; 1.1266x over previous
import jax
import jax.numpy as jnp
from jax import lax
from jax.experimental import pallas as pl
from jax.experimental.pallas import tpu as pltpu
from jax.experimental.pallas import tpu_sc as plsc

B = 16384
D = 64
VOCAB = 1000000
NBLK = (VOCAB + 127) // 128
NC = 2
NS = 16
NW = NC * NS
WIN = 512
OB = 1024
NBUF = 7

_mesh = plsc.VectorSubcoreMesh(core_axis_name="c", subcore_axis_name="s",
                               num_cores=NC, num_subcores=NS)


def _mul(x, n):
    return pl.multiple_of(x, n)


def _sc_body(xs_hbm, sids_hbm, bounds_hbm, Vt_hbm, out_hbm,
             swin, xwin, bv, buf0, buf1, buf2, buf3, buf4, buf5, buf6, S, obuf,
             sem0, sem1, sem2, sem3, sem4, sem5, sem6, osem):
    wid = lax.axis_index("s") * NC + lax.axis_index("c")
    qlo = (NBLK * wid) // NW
    qhi = (NBLK * (wid + 1)) // NW
    nq = qhi - qlo

    lanes = lax.iota(jnp.int32, 16)
    flanes = lanes.astype(jnp.float32) * 0.0
    zeros16 = jnp.zeros((16,), jnp.int32)
    fzeros = jnp.zeros((16,), jnp.float32)
    cvecs = [lanes + j * 16 for j in range(D // 16)]

    pltpu.sync_copy(bounds_hbm, bv)

    def sx(ref, pos):
        c = ref[pl.ds(_mul(lax.shift_left(lax.shift_right_logical(pos, 4), 4), 16), 16)]
        return jnp.sum(jnp.where(lanes == lax.bitwise_and(pos, 15), c, zeros16))

    lo = sx(bv, wid)
    hi = sx(bv, wid + 1)

    for k in range(OB // 16):
        obuf[pl.ds(k * 16, 16)] = fzeros
    zcps = [
        pltpu.async_copy(obuf, out_hbm.at[wid, pl.ds(k * OB, OB)], osem)
        for k in range(B // OB)
    ]
    for z in zcps:
        z.wait()

    win0 = _mul(jnp.minimum(lax.shift_left(lax.shift_right_logical(lo, 3), 3), B - WIN), 8)
    pltpu.sync_copy(sids_hbm.at[pl.ds(win0, WIN)], swin)
    pltpu.sync_copy(xs_hbm.at[pl.ds(win0, WIN)], xwin)

    bufs = (buf0, buf1, buf2, buf3, buf4, buf5, buf6)
    sems = (sem0, sem1, sem2, sem3, sem4, sem5, sem6)

    def fetch(qidx, b):
        qf = jnp.minimum(qidx, NBLK - 1)
        off = _mul(qf * 128, 128)
        pltpu.async_copy(Vt_hbm.at[:, pl.ds(off, 128)], bufs[b], sems[b])

    for b in range(NBUF):
        fetch(qlo + b, b)

    def wait(b):
        pltpu.make_async_copy(Vt_hbm.at[:, pl.ds(0, 128)], bufs[b], sems[b]).wait()

    def group_emit(p_last):
        g0 = lax.shift_left(lax.shift_right_logical(p_last, 4), 4)
        acc = plsc.load_gather(S, [lanes, zeros16])
        for cc in range(1, 16):
            acc = acc + plsc.load_gather(S, [lanes, zeros16 + cc])
        sig = 1.0 / (1.0 + jnp.exp(-acc))
        gpos = g0 + lanes
        vmask = jnp.logical_and(gpos >= lo, gpos < hi)
        sig = jnp.where(vmask, sig, flanes)
        obuf[pl.ds(_mul(lax.bitwise_and(g0, OB - 1), 16), 16)] = sig

    def flush(p_last):
        fb = _mul(lax.shift_left(lax.shift_right_logical(p_last, 10), 10), 1024)
        pltpu.async_copy(obuf, out_hbm.at[wid, pl.ds(fb, OB)], osem).wait()
        for k in range(OB // 16):
            obuf[pl.ds(k * 16, 16)] = fzeros

    def run_block(q, b, carry):
        buf = bufs[b]

        def cond(c):
            p, win, s = c
            return jnp.logical_and(p < hi, lax.shift_right_logical(s, 7) == q)

        def body(c):
            p, win, s = c
            off = p - win
            m = lax.bitwise_and(s, 127)
            msplat = zeros16 + m
            pv = plsc.load_gather(buf, [cvecs[0], msplat]) * xwin[off, pl.ds(0, 16)]
            for j in range(1, D // 16):
                pv = pv + plsc.load_gather(buf, [cvecs[j], msplat]) * xwin[off, pl.ds(j * 16, 16)]
            S[lax.bitwise_and(p, 15), pl.ds(0, 16)] = pv

            @pl.when(lax.bitwise_and(p, 15) == 15)
            def _():
                group_emit(p)

                @pl.when(lax.bitwise_and(p, OB - 1) == OB - 1)
                def _():
                    flush(p)

            p2 = p + 1
            need = (p2 - win) >= WIN
            nw = _mul(jnp.minimum(lax.shift_left(lax.shift_right_logical(p2, 3), 3), B - WIN), 8)

            @pl.when(need)
            def _():
                pltpu.sync_copy(sids_hbm.at[pl.ds(nw, WIN)], swin)
                pltpu.sync_copy(xs_hbm.at[pl.ds(nw, WIN)], xwin)

            win2 = jnp.where(need, nw, win)
            s2 = sx(swin, jnp.minimum(p2 - win2, WIN - 1))
            return (p2, win2, s2)

        return lax.while_loop(cond, body, carry)

    def outer(o, carry):
        for b in range(NBUF):
            k = o * NBUF + b
            wait(b)
            carry = run_block(qlo + k, b, carry)
            fetch(qlo + k + NBUF, b)
        return carry

    s0 = sx(swin, jnp.minimum(lo - win0, WIN - 1))
    nqo = lax.div(nq + NBUF - 1, jnp.int32(NBUF))
    p, win, _s = lax.fori_loop(0, nqo, outer, (lo, win0, s0))
    for b in range(NBUF):
        wait(b)

    @pl.when(jnp.logical_and(hi > lo, lax.bitwise_and(hi, 15) != 0))
    def _():
        group_emit(hi - 1)

    @pl.when(jnp.logical_and(hi > lo, lax.bitwise_and(hi, OB - 1) != 0))
    def _():
        flush(hi - 1)


_sc_kernel = pl.kernel(
    _sc_body,
    out_type=jax.ShapeDtypeStruct((NW, B), jnp.float32),
    mesh=_mesh,
    compiler_params=pltpu.CompilerParams(needs_layout_passes=False),
    scratch_types=[
        pltpu.VMEM((WIN,), jnp.int32),
        pltpu.VMEM((WIN, D), jnp.float32),
        pltpu.VMEM((64,), jnp.int32),
        pltpu.VMEM((D, 128), jnp.float32),
        pltpu.VMEM((D, 128), jnp.float32),
        pltpu.VMEM((D, 128), jnp.float32),
        pltpu.VMEM((D, 128), jnp.float32),
        pltpu.VMEM((D, 128), jnp.float32),
        pltpu.VMEM((D, 128), jnp.float32),
        pltpu.VMEM((D, 128), jnp.float32),
        pltpu.VMEM((16, 17), jnp.float32),
        pltpu.VMEM((OB,), jnp.float32),
        pltpu.SemaphoreType.DMA,
        pltpu.SemaphoreType.DMA,
        pltpu.SemaphoreType.DMA,
        pltpu.SemaphoreType.DMA,
        pltpu.SemaphoreType.DMA,
        pltpu.SemaphoreType.DMA,
        pltpu.SemaphoreType.DMA,
        pltpu.SemaphoreType.DMA,
    ],
)


@jax.jit
def kernel(x, emoji_ids, V):
    ids = emoji_ids.astype(jnp.int32)
    iot = jnp.arange(B, dtype=jnp.int32)
    sids, perm = lax.sort_key_val(ids, iot)
    _, inv_perm = lax.sort_key_val(perm, iot)
    x_s = jnp.take(x, perm, axis=0)
    qsplit = jnp.array([((NBLK * w) // NW) * 128 for w in range(NW + 1)],
                       dtype=jnp.int32)
    bounds = jnp.sum(qsplit[:, None] > sids[None, :], axis=1,
                      dtype=jnp.int32)
    boundsp = jnp.zeros((64,), jnp.int32).at[: NW + 1].set(bounds)
    out_p = _sc_kernel(x_s, sids, boundsp, V.T)
    out_s = jnp.sum(out_p, axis=0)
    return jnp.take(out_s, inv_perm)
